# baseline (speedup 1.0000x reference)
import jax
import jax.numpy as jnp
from jax import lax
from jax.experimental import pallas as pl
from jax.experimental.pallas import tpu as pltpu
from jax.experimental.pallas import tpu_sc as plsc

D_MODEL = 2048
B_TOTAL = 4 * 8192
NUM_CORES = 2
NUM_SUBCORES = 16
NW = NUM_CORES * NUM_SUBCORES
B_PER_W = B_TOTAL // NW
CHUNK = 8
NCH = B_PER_W // CHUNK
NBUF = 4
NGRP = NCH // NBUF


def _gather_body(idx_hbm, table_hbm, out_hbm, idx_v, *bufs_and_sems):
    bufs = bufs_and_sems[:NBUF]
    gsems = bufs_and_sems[NBUF:2 * NBUF]
    osems = bufs_and_sems[2 * NBUF:3 * NBUF]

    wid = lax.axis_index("s") * NUM_CORES + lax.axis_index("c")
    chunk0 = wid * NCH
    pltpu.sync_copy(idx_hbm.at[pl.ds(chunk0, NCH)], idx_v)

    def gstart(j, k):
        pltpu.async_copy(table_hbm.at[idx_v.at[j]], bufs[k], gsems[k])

    def gwait(j, k):
        pltpu.make_async_copy(table_hbm.at[idx_v.at[j]], bufs[k],
                              gsems[k]).wait()

    def ostart(j, k):
        pltpu.async_copy(bufs[k],
                         out_hbm.at[pl.ds((chunk0 + j) * CHUNK, CHUNK)],
                         osems[k])

    def owait(j, k):
        pltpu.make_async_copy(bufs[k],
                              out_hbm.at[pl.ds((chunk0 + j) * CHUNK, CHUNK)],
                              osems[k]).wait()

    for k in range(NBUF):
        gstart(k, k)

    def body(i, carry):
        j0 = i * NBUF
        for k in range(NBUF):
            gwait(j0 + k, k)
            ostart(j0 + k, k)
        for k in range(NBUF):
            owait(j0 + k, k)
            gstart(j0 + NBUF + k, k)
        return carry

    lax.fori_loop(0, NGRP - 1, body, 0)

    j0 = (NGRP - 1) * NBUF
    for k in range(NBUF):
        gwait(j0 + k, k)
        ostart(j0 + k, k)
    for k in range(NBUF):
        owait(j0 + k, k)


@jax.jit
def _run(idx2d, table):
    return pl.kernel(
        _gather_body,
        out_type=jax.ShapeDtypeStruct((B_TOTAL, D_MODEL), jnp.float32),
        mesh=plsc.VectorSubcoreMesh(core_axis_name="c", subcore_axis_name="s"),
        scratch_types=(
            [pltpu.VMEM((NCH, CHUNK), jnp.int32)]
            + [pltpu.VMEM((CHUNK, D_MODEL), jnp.float32)] * NBUF
            + [pltpu.SemaphoreType.DMA] * (2 * NBUF)
        ),
    )(idx2d, table)


def kernel(thought_ids, thought_embeddings):
    batch_shape = thought_ids.shape
    idx2d = jnp.asarray(thought_ids, jnp.int32).reshape(B_TOTAL // CHUNK, CHUNK)
    out = _run(idx2d, thought_embeddings)
    return out.reshape(*batch_shape, D_MODEL)


# speedup vs baseline: 1.0017x; 1.0017x over previous
import jax
import jax.numpy as jnp
from jax import lax
from jax.experimental import pallas as pl
from jax.experimental.pallas import tpu as pltpu
from jax.experimental.pallas import tpu_sc as plsc

D_MODEL = 2048
B_TOTAL = 4 * 8192
NUM_CORES = 2
NUM_SUBCORES = 16
NW = NUM_CORES * NUM_SUBCORES
B_PER_W = B_TOTAL // NW
CHUNK = 16
NCH = B_PER_W // CHUNK
NBUF = 3
NROUND = NCH // NBUF
NTAIL = NCH - NROUND * NBUF


def _gather_body(idx_hbm, table_hbm, out_hbm, idx_v, *bufs_and_sems):
    bufs = bufs_and_sems[:NBUF]
    gsems = bufs_and_sems[NBUF:2 * NBUF]
    osems = bufs_and_sems[2 * NBUF:3 * NBUF]

    wid = lax.axis_index("s") * NUM_CORES + lax.axis_index("c")
    chunk0 = wid * NCH
    pltpu.sync_copy(idx_hbm.at[pl.ds(chunk0, NCH)], idx_v)

    def gstart(j, k):
        pltpu.async_copy(table_hbm.at[idx_v.at[j]], bufs[k], gsems[k])

    def gwait(j, k):
        pltpu.make_async_copy(table_hbm.at[idx_v.at[j]], bufs[k],
                              gsems[k]).wait()

    def ostart(j, k):
        pltpu.async_copy(bufs[k],
                         out_hbm.at[pl.ds((chunk0 + j) * CHUNK, CHUNK)],
                         osems[k])

    def owait(j, k):
        pltpu.make_async_copy(bufs[k],
                              out_hbm.at[pl.ds((chunk0 + j) * CHUNK, CHUNK)],
                              osems[k]).wait()

    for k in range(NBUF):
        gstart(k, k)

    def body(i, carry):
        j0 = i * NBUF
        for k in range(NBUF):
            gwait(j0 + k, k)
            ostart(j0 + k, k)
        for k in range(NBUF):
            owait(j0 + k, k)
            gstart(j0 + NBUF + k, k)
        return carry

    lax.fori_loop(0, NROUND - 1, body, 0)

    j0 = (NROUND - 1) * NBUF
    for k in range(NBUF):
        gwait(j0 + k, k)
        ostart(j0 + k, k)
    for k in range(NTAIL):
        owait(j0 + k, k)
        gstart(j0 + NBUF + k, k)
    for k in range(NTAIL, NBUF):
        owait(j0 + k, k)
    j0 = NROUND * NBUF
    for k in range(NTAIL):
        gwait(j0 + k, k)
        ostart(j0 + k, k)
    for k in range(NTAIL):
        owait(j0 + k, k)


@jax.jit
def _run(idx2d, table):
    return pl.kernel(
        _gather_body,
        out_type=jax.ShapeDtypeStruct((B_TOTAL, D_MODEL), jnp.float32),
        mesh=plsc.VectorSubcoreMesh(core_axis_name="c", subcore_axis_name="s"),
        scratch_types=(
            [pltpu.VMEM((NCH, CHUNK), jnp.int32)]
            + [pltpu.VMEM((CHUNK, D_MODEL), jnp.float32)] * NBUF
            + [pltpu.SemaphoreType.DMA] * (2 * NBUF)
        ),
    )(idx2d, table)


def kernel(thought_ids, thought_embeddings):
    batch_shape = thought_ids.shape
    idx2d = jnp.asarray(thought_ids, jnp.int32).reshape(B_TOTAL // CHUNK, CHUNK)
    out = _run(idx2d, thought_embeddings)
    return out.reshape(*batch_shape, D_MODEL)
